# baseline (speedup 1.0000x reference)
import jax
import jax.numpy as jnp
from jax import lax
from jax.experimental import pallas as pl
from jax.experimental.pallas import tpu as pltpu
from jax.experimental.pallas import tpu_sc as plsc

VOCAB = 1000000
EMBED = 64
B = 16384
L = 50

NC = 2
NS = 16
NW = NC * NS

ROWS_W = B // NW
KR = 16
NGROUP = ROWS_W // KR


def _embed_body(weight_hbm, x_hbm, out_hbm, idx_v, rows_v, semg, sems):
    wid = lax.axis_index("s") * NC + lax.axis_index("c")
    base = wid * ROWS_W
    pltpu.sync_copy(x_hbm.at[pl.ds(base, ROWS_W)], idx_v)

    def fire_gathers(g, buf, sem):
        for j in range(KR):
            pltpu.async_copy(
                weight_hbm.at[idx_v.at[g * KR + j]],
                rows_v.at[buf, j],
                sem,
            )

    def drain_gathers(buf, sem):
        pltpu.make_async_copy(
            out_hbm.at[pl.ds(0, KR)], rows_v.at[buf], sem
        ).wait()

    def fire_scatter(g, buf, sem):
        pltpu.async_copy(
            rows_v.at[buf], out_hbm.at[pl.ds(base + g * KR, KR)], sem
        )

    def wait_scatter(g, buf, sem):
        pltpu.make_async_copy(
            rows_v.at[buf], out_hbm.at[pl.ds(base + g * KR, KR)], sem
        ).wait()

    fire_gathers(0, 0, semg.at[0])
    fire_gathers(1, 1, semg.at[1])
    drain_gathers(0, semg.at[0])
    fire_scatter(0, 0, sems.at[0])

    def pair(t, _):
        g = 2 * t + 1
        wait_scatter(g - 1, 0, sems.at[0])
        fire_gathers(g + 1, 0, semg.at[0])
        drain_gathers(1, semg.at[1])
        fire_scatter(g, 1, sems.at[1])
        wait_scatter(g, 1, sems.at[1])
        fire_gathers(g + 2, 1, semg.at[1])
        drain_gathers(0, semg.at[0])
        fire_scatter(g + 1, 0, sems.at[0])
        return 0

    lax.fori_loop(0, (NGROUP - 2) // 2, pair, 0)

    wait_scatter(NGROUP - 2, 0, sems.at[0])
    drain_gathers(1, semg.at[1])
    fire_scatter(NGROUP - 1, 1, sems.at[1])
    wait_scatter(NGROUP - 1, 1, sems.at[1])


@jax.jit
def kernel(x, weight):
    mesh = plsc.VectorSubcoreMesh(core_axis_name="c", subcore_axis_name="s")
    out = pl.kernel(
        _embed_body,
        out_type=jax.ShapeDtypeStruct((B, L, EMBED), jnp.float32),
        mesh=mesh,
        scratch_types=[
            pltpu.VMEM((ROWS_W, L), jnp.int32),
            pltpu.VMEM((2, KR, L, EMBED), jnp.float32),
            pltpu.SemaphoreType.DMA((2,)),
            pltpu.SemaphoreType.DMA((2,)),
        ],
        compiler_params=pltpu.CompilerParams(use_tc_tiling_on_sc=False),
    )(weight, x)
    return out
